# Initial kernel scaffold; baseline (speedup 1.0000x reference)
import jax
import jax.numpy as jnp
from jax.experimental import pallas as pl


def kernel(x, edge_index, batch, params):
    raise NotImplementedError("write your pallas kernel here")


# speedup vs baseline: 8.5192x; 8.5192x over previous
import functools

import jax
import jax.numpy as jnp
from jax import lax
from jax.experimental import pallas as pl
from jax.experimental.pallas import tpu as pltpu
from jax.experimental.pallas import tpu_sc as plsc

N = 50000
E = 800000
F_IN = 7
H = 128
H2 = 6 * H
G = 256
OUT = 3
EPS = 1e-5
NEG = 0.01

NC, NS, L = 2, 16, 16
NPAD = 50176
ZR = NPAD // NS
NROWS = 50688
BN_TC = 2000
ABN = 704
MBN = 400

EPR = 6336
EP = EPR * 128
NB_PAD = 53248
NBR = 416
BH_ACC = 512

_mesh = plsc.VectorSubcoreMesh(core_axis_name="c", subcore_axis_name="s")
_scp = pltpu.CompilerParams(use_tc_tiling_on_sc=False,
                            needs_layout_passes=False)


def _make_hist(nrows_per_tile, acc_rows):
    zr = acc_rows // NS

    @functools.partial(
        pl.kernel, mesh=_mesh, compiler_params=_scp,
        out_type=jax.ShapeDtypeStruct((NC, acc_rows, 16), jnp.float32),
        scratch_types=[
            pltpu.VMEM((128, 16), jnp.float32),
            pltpu.VMEM((nrows_per_tile, 2, 128), jnp.int32),
            pltpu.VMEM_SHARED((acc_rows, 16), jnp.float32),
            pltpu.SemaphoreType.DMA,
        ],
    )
    def hist(ids_hbm, ones_hbm, z_hbm, out_hbm, ones_v, idx_v, acc, ssem):
        cid = lax.axis_index("c")
        sid = lax.axis_index("s")
        pltpu.sync_copy(ones_hbm, ones_v)
        nz = zr // 784 if zr % 784 == 0 else 0
        if nz:
            for k in range(nz):
                pltpu.sync_copy(
                    z_hbm, acc.at[pl.ds(sid * zr + k * 784, 784)])
        else:
            pltpu.sync_copy(z_hbm.at[pl.ds(0, zr)],
                            acc.at[pl.ds(sid * zr, zr)])
        row0 = cid * (NS * nrows_per_tile) + sid * nrows_per_tile
        pltpu.sync_copy(ids_hbm.at[pl.ds(row0, nrows_per_tile)], idx_v)
        plsc.subcore_barrier()
        if nrows_per_tile % 6 == 0:
            def body(w, _):
                d = [pltpu.async_copy(ones_v, acc.at[idx_v.at[w * 6 + j, 1]],
                                      ssem, add=True)
                     for j in range(6)]
                for x in d:
                    x.wait()
                return 0

            lax.fori_loop(0, nrows_per_tile // 6, body, 0)
        else:
            d = [pltpu.async_copy(ones_v, acc.at[idx_v.at[j, 1]], ssem,
                                  add=True)
                 for j in range(nrows_per_tile)]
            for x in d:
                x.wait()
        plsc.subcore_barrier()
        pltpu.sync_copy(acc.at[pl.ds(sid * zr, zr)],
                        out_hbm.at[cid, pl.ds(sid * zr, zr), :])

    return hist


_hist_deg = _make_hist(198, NPAD)
_hist_batch = _make_hist(13, BH_ACC)


def _edge_pass(h_hbm, e_hbm, acc, eidx, r0, r1, gsem, ssem, row0, nbod):
    def fire_g(rb, j0):
        return [pltpu.async_copy(h_hbm.at[eidx.at[j0 + j, 0]],
                                 rb.at[pl.ds(j * 128, 128)], gsem)
                for j in range(3)]

    def fire_s(rb, j0):
        return [pltpu.async_copy(rb.at[pl.ds(j * 128, 128)],
                                 acc.at[eidx.at[j0 + j, 1]], ssem,
                                 add=True)
                for j in range(3)]

    def body(i, _):
        pltpu.sync_copy(e_hbm.at[pl.ds(row0 + i * 6, 6)], eidx)
        ga = fire_g(r0, 0)
        for d in ga:
            d.wait()
        sa = fire_s(r0, 0)
        gb = fire_g(r1, 3)
        for d in gb:
            d.wait()
        for d in sa:
            d.wait()
        sb = fire_s(r1, 3)
        for d in sb:
            d.wait()
        return 0

    lax.fori_loop(0, nbod, body, 0)


@functools.partial(
    pl.kernel, mesh=_mesh, compiler_params=_scp,
    out_type=jax.ShapeDtypeStruct((NC, NPAD, 16), jnp.float32),
    scratch_types=[
        pltpu.VMEM((6, 2, 128), jnp.int32),
        pltpu.VMEM((384, 16), jnp.float32),
        pltpu.VMEM((384, 16), jnp.float32),
        pltpu.VMEM_SHARED((NPAD, 16), jnp.float32),
        pltpu.SemaphoreType.DMA,
        pltpu.SemaphoreType.DMA,
    ],
)
def _hop16(h_hbm, e_hbm, z_hbm, out_hbm, eidx, r0, r1, acc, gsem, ssem):
    cid = lax.axis_index("c")
    sid = lax.axis_index("s")
    for k in range(4):
        pltpu.sync_copy(z_hbm, acc.at[pl.ds(sid * ZR + k * 784, 784)])
    plsc.subcore_barrier()
    row0 = cid * (NS * 198) + sid * 198
    _edge_pass(h_hbm, e_hbm, acc, eidx, r0, r1, gsem, ssem, row0, 33)
    plsc.subcore_barrier()
    pltpu.sync_copy(acc.at[pl.ds(sid * ZR, ZR)],
                    out_hbm.at[cid, pl.ds(sid * ZR, ZR), :])


@functools.partial(
    pl.kernel, mesh=_mesh, compiler_params=_scp,
    out_type=jax.ShapeDtypeStruct((4, NPAD, 32), jnp.float32),
    scratch_types=[
        pltpu.VMEM((6, 2, 128), jnp.int32),
        pltpu.VMEM((384, 32), jnp.float32),
        pltpu.VMEM((384, 32), jnp.float32),
        pltpu.VMEM_SHARED((NPAD, 32), jnp.float32),
        pltpu.SemaphoreType.DMA,
        pltpu.SemaphoreType.DMA,
    ],
)
def _hop32(h0, h1, h2, h3, e_hbm, z_hbm, out_hbm, eidx, r0, r1, acc,
           gsem, ssem):
    cid = lax.axis_index("c")
    sid = lax.axis_index("s")
    hs = (h0, h1, h2, h3)
    for q in range(4):
        @pl.when(cid == q // 2)
        def _q(q=q):
            for k in range(4):
                pltpu.sync_copy(z_hbm,
                                acc.at[pl.ds(sid * ZR + k * 784, 784)])
            plsc.subcore_barrier()
            _edge_pass(hs[q], e_hbm, acc, eidx, r0, r1, gsem, ssem,
                       sid * 396, 66)
            plsc.subcore_barrier()
            pltpu.sync_copy(acc.at[pl.ds(sid * ZR, ZR)],
                            out_hbm.at[q, pl.ds(sid * ZR, ZR), :])
            plsc.subcore_barrier()


RW = 256

@functools.partial(
    pl.kernel, mesh=_mesh, compiler_params=_scp,
    out_type=jax.ShapeDtypeStruct((G * 256,), jnp.float32),
    scratch_types=[
        pltpu.VMEM((RW, 32), jnp.float32),
        pltpu.VMEM((272,), jnp.int32),
        pltpu.VMEM((8 * 256,), jnp.float32),
    ],
)
def _pool(h0, h1, h2, h3, bnd_hbm, out_hbm, buf_v, bnd_v, res_v):
    cid = lax.axis_index("c")
    sid = lax.axis_index("s")
    wid = sid * NC + cid
    pltpu.sync_copy(bnd_hbm, bnd_v)
    lanes = lax.iota(jnp.int32, 16)
    g0 = wid * 8
    sv = plsc.load_gather(bnd_v, [g0 + lanes])
    ev = plsc.load_gather(bnd_v, [g0 + 1 + lanes])
    hs = (h0, h1, h2, h3)
    for gi in range(8):
        s = sv[gi]
        e = ev[gi]
        cnt = e - s
        nwin = (cnt + RW - 1) // RW
        denom = jnp.maximum(cnt.astype(jnp.float32), 1.0)
        for q in range(4):
            hq = hs[q]

            def wbody(w, carry, hq=hq, s=s, cnt=cnt):
                s0, s1, m0, m1 = carry
                ws = s + w * RW
                pltpu.sync_copy(hq.at[pl.ds(ws, RW)], buf_v)
                rem = jnp.minimum(RW, cnt - w * RW)

                def rbody(r, c):
                    a0, a1, b0, b1 = c
                    v0 = buf_v[r, pl.ds(0, 16)]
                    v1 = buf_v[r, pl.ds(16, 16)]
                    return (a0 + v0, a1 + v1,
                            jnp.maximum(b0, v0), jnp.maximum(b1, v1))

                return lax.fori_loop(0, rem, rbody, (s0, s1, m0, m1))

            init = (jnp.zeros((16,), jnp.float32),
                    jnp.zeros((16,), jnp.float32),
                    jnp.full((16,), -jnp.inf, jnp.float32),
                    jnp.full((16,), -jnp.inf, jnp.float32))
            s0, s1, m0, m1 = lax.fori_loop(0, nwin, wbody, init)
            base = gi * 256 + q * 32
            res_v[pl.ds(base, 16)] = s0 / denom
            res_v[pl.ds(base + 16, 16)] = s1 / denom
            res_v[pl.ds(base + 128, 16)] = jnp.where(cnt > 0, m0, 0.0)
            res_v[pl.ds(base + 144, 16)] = jnp.where(cnt > 0, m1, 0.0)
    pltpu.sync_copy(res_v, out_hbm.at[pl.ds(wid * 2048, 2048)])


def _leaky(x):
    return jnp.where(x > 0, x, NEG * x)


def _prep_body(x_ref, d0_ref, d1_ref, dinv_ref, g_ref):
    deg = d0_ref[0, :, 0:1] + d1_ref[0, :, 0:1]
    dinv = jnp.where(deg > 0, lax.rsqrt(jnp.maximum(deg, 1.0)), 0.0)
    dinv_ref[...] = dinv
    g_ref[...] = jnp.concatenate(
        [x_ref[...] * dinv, jnp.zeros((BN_TC, 16 - F_IN), jnp.float32)],
        axis=1)


def _tc_prep(x, degp):
    return pl.pallas_call(
        _prep_body,
        grid=(N // BN_TC,),
        in_specs=[
            pl.BlockSpec((BN_TC, F_IN), lambda i: (i, 0)),
            pl.BlockSpec((1, BN_TC, 16), lambda i: (0, i, 0)),
            pl.BlockSpec((1, BN_TC, 16), lambda i: (1, i, 0)),
        ],
        out_specs=[
            pl.BlockSpec((BN_TC, 1), lambda i: (i, 0)),
            pl.BlockSpec((BN_TC, 16), lambda i: (i, 0)),
        ],
        out_shape=[
            jax.ShapeDtypeStruct((N, 1), jnp.float32),
            jax.ShapeDtypeStruct((N, 16), jnp.float32),
        ],
    )(x, degp, degp)


def _l1mid_body(s0_ref, s1_ref, dinv_ref, p1_ref, g2_ref):
    dinv = dinv_ref[...]
    p1 = (s0_ref[0] + s1_ref[0]) * dinv
    p1_ref[...] = p1
    g2_ref[...] = p1 * dinv


def _tc_l1mid(s1p, dinv):
    return pl.pallas_call(
        _l1mid_body,
        grid=(N // BN_TC,),
        in_specs=[
            pl.BlockSpec((1, BN_TC, 16), lambda i: (0, i, 0)),
            pl.BlockSpec((1, BN_TC, 16), lambda i: (1, i, 0)),
            pl.BlockSpec((BN_TC, 1), lambda i: (i, 0)),
        ],
        out_specs=[
            pl.BlockSpec((BN_TC, 16), lambda i: (i, 0)),
            pl.BlockSpec((BN_TC, 16), lambda i: (i, 0)),
        ],
        out_shape=[
            jax.ShapeDtypeStruct((N, 16), jnp.float32),
            jax.ShapeDtypeStruct((N, 16), jnp.float32),
        ],
    )(s1p, s1p, dinv)


def _act1_body(x_ref, p1_ref, s20_ref, s21_ref, dinv_ref, w0_ref, w1_ref,
               w2_ref, b_ref, h0, h1, h2, h3, g0, g1, g2, g3):
    dinv = dinv_ref[...]
    p2 = (s20_ref[0] + s21_ref[0]) * dinv
    acc = (jnp.dot(x_ref[...], w0_ref[...]) +
           jnp.dot(p1_ref[...], w1_ref[...]) +
           jnp.dot(p2, w2_ref[...]) + b_ref[...])
    h = _leaky(acc)
    g = h * dinv
    for q, (hr, gr) in enumerate(((h0, g0), (h1, g1), (h2, g2), (h3, g3))):
        hr[...] = h[:, q * 32:(q + 1) * 32]
        gr[...] = g[:, q * 32:(q + 1) * 32]


def _tc_act1(x, p1, s2p, dinv, w0, w1p, w2p, b):
    hsp = [jax.ShapeDtypeStruct((NROWS, 32), jnp.float32)] * 4
    gsp = [jax.ShapeDtypeStruct((N, 32), jnp.float32)] * 4
    row = lambda i: (i, 0)
    return pl.pallas_call(
        _act1_body,
        grid=(NROWS // ABN,),
        in_specs=[
            pl.BlockSpec((ABN, F_IN), row),
            pl.BlockSpec((ABN, 16), row),
            pl.BlockSpec((1, ABN, 16), lambda i: (0, i, 0)),
            pl.BlockSpec((1, ABN, 16), lambda i: (1, i, 0)),
            pl.BlockSpec((ABN, 1), row),
            pl.BlockSpec((F_IN, H), lambda i: (0, 0)),
            pl.BlockSpec((16, H), lambda i: (0, 0)),
            pl.BlockSpec((16, H), lambda i: (0, 0)),
            pl.BlockSpec((1, H), lambda i: (0, 0)),
        ],
        out_specs=[pl.BlockSpec((ABN, 32), row)] * 8,
        out_shape=hsp + gsp,
    )(x, p1, s2p, s2p, dinv, w0, w1p, w2p, b)


def _mid_body(s0_ref, s1_ref, s2_ref, s3_ref, dinv_ref,
              p0, p1, p2, p3, g0, g1, g2, g3):
    dinv = dinv_ref[...]
    for sr, pr, gr in ((s0_ref, p0, g0), (s1_ref, p1, g1),
                       (s2_ref, p2, g2), (s3_ref, p3, g3)):
        p = sr[0] * dinv
        pr[...] = p
        gr[...] = p * dinv


def _tc_mid(sq, dinv):
    row = lambda i: (i, 0)
    qspec = [pl.BlockSpec((1, MBN, 32), (lambda q: (lambda i: (q, i, 0)))(q))
             for q in range(4)]
    osp = [jax.ShapeDtypeStruct((N, 32), jnp.float32)] * 8
    return pl.pallas_call(
        _mid_body,
        grid=(N // MBN,),
        in_specs=qspec + [pl.BlockSpec((MBN, 1), row)],
        out_specs=[pl.BlockSpec((MBN, 32), row)] * 8,
        out_shape=osp,
    )(sq, sq, sq, sq, dinv)


def _act23_body(h0r, h1r, h2r, h3r, p0r, p1r, p2r, p3r,
                s0r, s1r, s2r, s3r, dinv_ref, w0_ref, w1_ref, w2_ref, b_ref,
                h0, h1, h2, h3, g0, g1, g2, g3):
    dinv = dinv_ref[...]
    hprev = jnp.concatenate([h0r[...], h1r[...], h2r[...], h3r[...]], axis=1)
    p1 = jnp.concatenate([p0r[...], p1r[...], p2r[...], p3r[...]], axis=1)
    p2 = jnp.concatenate([s0r[0], s1r[0], s2r[0], s3r[0]], axis=1) * dinv
    acc = (jnp.dot(hprev, w0_ref[...]) + jnp.dot(p1, w1_ref[...]) +
           jnp.dot(p2, w2_ref[...]) + b_ref[...])
    h = _leaky(acc)
    g = h * dinv
    for q, (hr, gr) in enumerate(((h0, g0), (h1, g1), (h2, g2), (h3, g3))):
        hr[...] = h[:, q * 32:(q + 1) * 32]
        gr[...] = g[:, q * 32:(q + 1) * 32]


def _tc_act23(hq, p1q, s2q, dinv, w0, w1, w2, b):
    row = lambda i: (i, 0)
    qspec = [pl.BlockSpec((1, ABN, 32),
                          (lambda q: (lambda i: (q, i, 0)))(q))
             for q in range(4)]
    hsp = [jax.ShapeDtypeStruct((NROWS, 32), jnp.float32)] * 4
    gsp = [jax.ShapeDtypeStruct((N, 32), jnp.float32)] * 4
    return pl.pallas_call(
        _act23_body,
        grid=(NROWS // ABN,),
        in_specs=(
            [pl.BlockSpec((ABN, 32), row)] * 8 +
            qspec +
            [pl.BlockSpec((ABN, 1), row),
             pl.BlockSpec((H, H), lambda i: (0, 0)),
             pl.BlockSpec((H, H), lambda i: (0, 0)),
             pl.BlockSpec((H, H), lambda i: (0, 0)),
             pl.BlockSpec((1, H), lambda i: (0, 0))]),
        out_specs=[pl.BlockSpec((ABN, 32), row)] * 8,
        out_shape=hsp + gsp,
    )(*hq, *p1q, s2q, s2q, s2q, s2q, dinv, w0, w1, w2, b)


def _head_body(x1r, x2r, x3r, gref, bref, w1, b1, w2, b2, w3, b3, w4, b4,
               w5, b5, wo, bo, out_ref):
    z = jnp.concatenate([x1r[...], x2r[...], x3r[...]], axis=1)
    mu = jnp.mean(z, axis=0, keepdims=True)
    var = jnp.mean((z - mu) * (z - mu), axis=0, keepdims=True)
    z = (z - mu) / jnp.sqrt(var + EPS) * gref[...] + bref[...]
    for w, b in ((w1, b1), (w2, b2), (w3, b3), (w4, b4), (w5, b5)):
        z = _leaky(jnp.dot(z, w[...]) + b[...])
    zo = jnp.dot(z, wo[...]) + bo[...]
    out_ref[...] = jnp.concatenate([jnp.tanh(zo[:, :2]), zo[:, 2:]], axis=1)


def _tc_head(x1, x2, x3, p):
    args = [x1, x2, x3, p['bn_g'].reshape(1, H2), p['bn_b'].reshape(1, H2)]
    for i in range(1, 6):
        args += [p['l%d_W' % i], p['l%d_b' % i].reshape(1, H2)]
    args += [p['out_W'], p['out_b'].reshape(1, OUT)]
    return pl.pallas_call(
        _head_body,
        out_shape=jax.ShapeDtypeStruct((G, OUT), jnp.float32),
    )(*args)


def kernel(x, edge_index, batch, params):
    p = params
    f32 = jnp.float32

    src = edge_index[0]
    dst = edge_index[1]
    pad_e = EP - E
    src_p = jnp.concatenate([src, (jnp.arange(pad_e, dtype=jnp.int32) % N)])
    dst_p = jnp.concatenate(
        [dst, N + (jnp.arange(pad_e, dtype=jnp.int32) % (NPAD - N))])
    edges2 = jnp.stack([src_p.reshape(EPR, 128), dst_p.reshape(EPR, 128)],
                       axis=1)

    batch_p = jnp.concatenate(
        [batch, jnp.full((NB_PAD - N,), G, jnp.int32)]).reshape(NBR, 1, 128)
    batch_p = jnp.concatenate([batch_p, batch_p], axis=1)

    ones16 = jnp.ones((128, 16), f32)
    z784 = jnp.zeros((784, 32), f32)
    z16 = jnp.zeros((784, 16), f32)

    degp = _hist_deg(edges2, ones16, z16)
    bhp = _hist_batch(batch_p, ones16, z16)

    counts = (bhp[0, :G, 0] + bhp[1, :G, 0]).astype(jnp.int32)
    bounds = jnp.concatenate([
        jnp.zeros((1,), jnp.int32), jnp.cumsum(counts),
        jnp.full((272 - G - 1,), N, jnp.int32)])

    dinv, g16 = _tc_prep(x, degp)

    s1p = _hop16(g16, edges2, z16)
    p1_16, g2_16 = _tc_l1mid(s1p, dinv)
    s2p = _hop16(g2_16, edges2, z16)

    w1p = jnp.zeros((16, H), f32).at[:F_IN].set(p['c1_W'][1])
    w2p = jnp.zeros((16, H), f32).at[:F_IN].set(p['c1_W'][2])
    outs = _tc_act1(x, p1_16, s2p, dinv, p['c1_W'][0], w1p, w2p,
                    p['c1_b'].reshape(1, H))
    hq, gq = list(outs[:4]), list(outs[4:])

    xs = []
    for layer in (2, 3):
        W = p['c%d_W' % layer]
        b = p['c%d_b' % layer].reshape(1, H)
        x_pool = _pool(*hq, bounds).reshape(G, 256)
        xs.append(x_pool)
        s1q = _hop32(*gq, edges2, z784)
        outs = _tc_mid(s1q, dinv)
        p1q, g2q = list(outs[:4]), list(outs[4:])
        s2q = _hop32(*g2q, edges2, z784)
        outs = _tc_act23(hq, p1q, s2q, dinv, W[0], W[1], W[2], b)
        hq, gq = list(outs[:4]), list(outs[4:])
    xs.append(_pool(*hq, bounds).reshape(G, 256))

    return _tc_head(xs[0], xs[1], xs[2], p)


# speedup vs baseline: 8.8764x; 1.0419x over previous
import functools

import jax
import jax.numpy as jnp
from jax import lax
from jax.experimental import pallas as pl
from jax.experimental.pallas import tpu as pltpu
from jax.experimental.pallas import tpu_sc as plsc

N = 50000
E = 800000
F_IN = 7
H = 128
H2 = 6 * H
G = 256
OUT = 3
EPS = 1e-5
NEG = 0.01

NC, NS, L = 2, 16, 16
NPAD = 50176
ZR = NPAD // NS
NROWS = 50688
BN_TC = 2000
ABN = 704
MBN = 400

EPR = 6272
EP = EPR * 128
NB_PAD = 53248
NBR = 416
BH_ACC = 512

_mesh = plsc.VectorSubcoreMesh(core_axis_name="c", subcore_axis_name="s")
_scp = pltpu.CompilerParams(use_tc_tiling_on_sc=False,
                            needs_layout_passes=False)


def _make_hist(nrows_per_tile, acc_rows):
    zr = acc_rows // NS

    @functools.partial(
        pl.kernel, mesh=_mesh, compiler_params=_scp,
        out_type=jax.ShapeDtypeStruct((NC, acc_rows, 16), jnp.float32),
        scratch_types=[
            pltpu.VMEM((128, 16), jnp.float32),
            pltpu.VMEM((nrows_per_tile, 2, 128), jnp.int32),
            pltpu.VMEM_SHARED((acc_rows, 16), jnp.float32),
            pltpu.SemaphoreType.DMA,
        ],
    )
    def hist(ids_hbm, ones_hbm, z_hbm, out_hbm, ones_v, idx_v, acc, ssem):
        cid = lax.axis_index("c")
        sid = lax.axis_index("s")
        pltpu.sync_copy(ones_hbm, ones_v)
        nz = zr // 784 if zr % 784 == 0 else 0
        if nz:
            for k in range(nz):
                pltpu.sync_copy(
                    z_hbm, acc.at[pl.ds(sid * zr + k * 784, 784)])
        else:
            pltpu.sync_copy(z_hbm.at[pl.ds(0, zr)],
                            acc.at[pl.ds(sid * zr, zr)])
        row0 = cid * (NS * nrows_per_tile) + sid * nrows_per_tile
        pltpu.sync_copy(ids_hbm.at[pl.ds(row0, nrows_per_tile)], idx_v)
        plsc.subcore_barrier()
        if nrows_per_tile % 7 == 0:
            def body(w, _):
                d = [pltpu.async_copy(ones_v, acc.at[idx_v.at[w * 7 + j, 1]],
                                      ssem, add=True)
                     for j in range(7)]
                for x in d:
                    x.wait()
                return 0

            lax.fori_loop(0, nrows_per_tile // 7, body, 0)
        else:
            d = [pltpu.async_copy(ones_v, acc.at[idx_v.at[j, 1]], ssem,
                                  add=True)
                 for j in range(nrows_per_tile)]
            for x in d:
                x.wait()
        plsc.subcore_barrier()
        pltpu.sync_copy(acc.at[pl.ds(sid * zr, zr)],
                        out_hbm.at[cid, pl.ds(sid * zr, zr), :])

    return hist


_hist_deg = _make_hist(196, NPAD)
_hist_batch = _make_hist(13, BH_ACC)


def _edge_pass(h_hbm, e_hbm, acc, eidx, r0, r1, gsem, ssem, row0, nbod):
    def fire_g(rb, j0, n):
        return [pltpu.async_copy(h_hbm.at[eidx.at[j0 + j, 0]],
                                 rb.at[pl.ds(j * 128, 128)], gsem)
                for j in range(n)]

    def fire_s(rb, j0, n):
        return [pltpu.async_copy(rb.at[pl.ds(j * 128, 128)],
                                 acc.at[eidx.at[j0 + j, 1]], ssem,
                                 add=True)
                for j in range(n)]

    def body(i, _):
        pltpu.sync_copy(e_hbm.at[pl.ds(row0 + i * 7, 7)], eidx)
        ga = fire_g(r0, 0, 4)
        for d in ga:
            d.wait()
        sa = fire_s(r0, 0, 4)
        gb = fire_g(r1, 4, 3)
        for d in gb:
            d.wait()
        for d in sa:
            d.wait()
        sb = fire_s(r1, 4, 3)
        for d in sb:
            d.wait()
        return 0

    lax.fori_loop(0, nbod, body, 0)


@functools.partial(
    pl.kernel, mesh=_mesh, compiler_params=_scp,
    out_type=jax.ShapeDtypeStruct((NC, NPAD, 16), jnp.float32),
    scratch_types=[
        pltpu.VMEM((7, 2, 128), jnp.int32),
        pltpu.VMEM((512, 16), jnp.float32),
        pltpu.VMEM((384, 16), jnp.float32),
        pltpu.VMEM_SHARED((NPAD, 16), jnp.float32),
        pltpu.SemaphoreType.DMA,
        pltpu.SemaphoreType.DMA,
    ],
)
def _hop16(h_hbm, e_hbm, z_hbm, out_hbm, eidx, r0, r1, acc, gsem, ssem):
    cid = lax.axis_index("c")
    sid = lax.axis_index("s")
    for k in range(4):
        pltpu.sync_copy(z_hbm, acc.at[pl.ds(sid * ZR + k * 784, 784)])
    plsc.subcore_barrier()
    row0 = cid * (NS * 196) + sid * 196
    _edge_pass(h_hbm, e_hbm, acc, eidx, r0, r1, gsem, ssem, row0, 28)
    plsc.subcore_barrier()
    pltpu.sync_copy(acc.at[pl.ds(sid * ZR, ZR)],
                    out_hbm.at[cid, pl.ds(sid * ZR, ZR), :])


@functools.partial(
    pl.kernel, mesh=_mesh, compiler_params=_scp,
    out_type=jax.ShapeDtypeStruct((4, NPAD, 32), jnp.float32),
    scratch_types=[
        pltpu.VMEM((7, 2, 128), jnp.int32),
        pltpu.VMEM((512, 32), jnp.float32),
        pltpu.VMEM((384, 32), jnp.float32),
        pltpu.VMEM_SHARED((NPAD, 32), jnp.float32),
        pltpu.SemaphoreType.DMA,
        pltpu.SemaphoreType.DMA,
    ],
)
def _hop32(h0, h1, h2, h3, e_hbm, z_hbm, out_hbm, eidx, r0, r1, acc,
           gsem, ssem):
    cid = lax.axis_index("c")
    sid = lax.axis_index("s")
    hs = (h0, h1, h2, h3)
    for q in range(4):
        @pl.when(cid == q // 2)
        def _q(q=q):
            for k in range(4):
                pltpu.sync_copy(z_hbm,
                                acc.at[pl.ds(sid * ZR + k * 784, 784)])
            plsc.subcore_barrier()
            _edge_pass(hs[q], e_hbm, acc, eidx, r0, r1, gsem, ssem,
                       sid * 392, 56)
            plsc.subcore_barrier()
            pltpu.sync_copy(acc.at[pl.ds(sid * ZR, ZR)],
                            out_hbm.at[q, pl.ds(sid * ZR, ZR), :])
            plsc.subcore_barrier()


RW = 256

@functools.partial(
    pl.kernel, mesh=_mesh, compiler_params=_scp,
    out_type=jax.ShapeDtypeStruct((G * 256,), jnp.float32),
    scratch_types=[
        pltpu.VMEM((RW, 32), jnp.float32),
        pltpu.VMEM((272,), jnp.int32),
        pltpu.VMEM((8 * 256,), jnp.float32),
    ],
)
def _pool(h0, h1, h2, h3, bnd_hbm, out_hbm, buf_v, bnd_v, res_v):
    cid = lax.axis_index("c")
    sid = lax.axis_index("s")
    wid = sid * NC + cid
    pltpu.sync_copy(bnd_hbm, bnd_v)
    lanes = lax.iota(jnp.int32, 16)
    g0 = wid * 8
    sv = plsc.load_gather(bnd_v, [g0 + lanes])
    ev = plsc.load_gather(bnd_v, [g0 + 1 + lanes])
    hs = (h0, h1, h2, h3)
    for gi in range(8):
        s = sv[gi]
        e = ev[gi]
        cnt = e - s
        nwin = (cnt + RW - 1) // RW
        denom = jnp.maximum(cnt.astype(jnp.float32), 1.0)
        for q in range(4):
            hq = hs[q]

            def wbody(w, carry, hq=hq, s=s, cnt=cnt):
                s0, s1, m0, m1 = carry
                ws = s + w * RW
                pltpu.sync_copy(hq.at[pl.ds(ws, RW)], buf_v)
                rem = jnp.minimum(RW, cnt - w * RW)

                def rbody(r, c):
                    a0, a1, b0, b1 = c
                    v0 = buf_v[r, pl.ds(0, 16)]
                    v1 = buf_v[r, pl.ds(16, 16)]
                    return (a0 + v0, a1 + v1,
                            jnp.maximum(b0, v0), jnp.maximum(b1, v1))

                return lax.fori_loop(0, rem, rbody, (s0, s1, m0, m1))

            init = (jnp.zeros((16,), jnp.float32),
                    jnp.zeros((16,), jnp.float32),
                    jnp.full((16,), -jnp.inf, jnp.float32),
                    jnp.full((16,), -jnp.inf, jnp.float32))
            s0, s1, m0, m1 = lax.fori_loop(0, nwin, wbody, init)
            base = gi * 256 + q * 32
            res_v[pl.ds(base, 16)] = s0 / denom
            res_v[pl.ds(base + 16, 16)] = s1 / denom
            res_v[pl.ds(base + 128, 16)] = jnp.where(cnt > 0, m0, 0.0)
            res_v[pl.ds(base + 144, 16)] = jnp.where(cnt > 0, m1, 0.0)
    pltpu.sync_copy(res_v, out_hbm.at[pl.ds(wid * 2048, 2048)])


def _leaky(x):
    return jnp.where(x > 0, x, NEG * x)


def _prep_body(x_ref, d0_ref, d1_ref, dinv_ref, g_ref):
    deg = d0_ref[0, :, 0:1] + d1_ref[0, :, 0:1]
    dinv = jnp.where(deg > 0, lax.rsqrt(jnp.maximum(deg, 1.0)), 0.0)
    dinv_ref[...] = dinv
    g_ref[...] = jnp.concatenate(
        [x_ref[...] * dinv, jnp.zeros((BN_TC, 16 - F_IN), jnp.float32)],
        axis=1)


def _tc_prep(x, degp):
    return pl.pallas_call(
        _prep_body,
        grid=(N // BN_TC,),
        in_specs=[
            pl.BlockSpec((BN_TC, F_IN), lambda i: (i, 0)),
            pl.BlockSpec((1, BN_TC, 16), lambda i: (0, i, 0)),
            pl.BlockSpec((1, BN_TC, 16), lambda i: (1, i, 0)),
        ],
        out_specs=[
            pl.BlockSpec((BN_TC, 1), lambda i: (i, 0)),
            pl.BlockSpec((BN_TC, 16), lambda i: (i, 0)),
        ],
        out_shape=[
            jax.ShapeDtypeStruct((N, 1), jnp.float32),
            jax.ShapeDtypeStruct((N, 16), jnp.float32),
        ],
    )(x, degp, degp)


def _l1mid_body(s0_ref, s1_ref, dinv_ref, p1_ref, g2_ref):
    dinv = dinv_ref[...]
    p1 = (s0_ref[0] + s1_ref[0]) * dinv
    p1_ref[...] = p1
    g2_ref[...] = p1 * dinv


def _tc_l1mid(s1p, dinv):
    return pl.pallas_call(
        _l1mid_body,
        grid=(N // BN_TC,),
        in_specs=[
            pl.BlockSpec((1, BN_TC, 16), lambda i: (0, i, 0)),
            pl.BlockSpec((1, BN_TC, 16), lambda i: (1, i, 0)),
            pl.BlockSpec((BN_TC, 1), lambda i: (i, 0)),
        ],
        out_specs=[
            pl.BlockSpec((BN_TC, 16), lambda i: (i, 0)),
            pl.BlockSpec((BN_TC, 16), lambda i: (i, 0)),
        ],
        out_shape=[
            jax.ShapeDtypeStruct((N, 16), jnp.float32),
            jax.ShapeDtypeStruct((N, 16), jnp.float32),
        ],
    )(s1p, s1p, dinv)


def _act1_body(x_ref, p1_ref, s20_ref, s21_ref, dinv_ref, w0_ref, w1_ref,
               w2_ref, b_ref, h0, h1, h2, h3, g0, g1, g2, g3):
    dinv = dinv_ref[...]
    p2 = (s20_ref[0] + s21_ref[0]) * dinv
    acc = (jnp.dot(x_ref[...], w0_ref[...]) +
           jnp.dot(p1_ref[...], w1_ref[...]) +
           jnp.dot(p2, w2_ref[...]) + b_ref[...])
    h = _leaky(acc)
    g = h * dinv
    for q, (hr, gr) in enumerate(((h0, g0), (h1, g1), (h2, g2), (h3, g3))):
        hr[...] = h[:, q * 32:(q + 1) * 32]
        gr[...] = g[:, q * 32:(q + 1) * 32]


def _tc_act1(x, p1, s2p, dinv, w0, w1p, w2p, b):
    hsp = [jax.ShapeDtypeStruct((NROWS, 32), jnp.float32)] * 4
    gsp = [jax.ShapeDtypeStruct((N, 32), jnp.float32)] * 4
    row = lambda i: (i, 0)
    return pl.pallas_call(
        _act1_body,
        grid=(NROWS // ABN,),
        in_specs=[
            pl.BlockSpec((ABN, F_IN), row),
            pl.BlockSpec((ABN, 16), row),
            pl.BlockSpec((1, ABN, 16), lambda i: (0, i, 0)),
            pl.BlockSpec((1, ABN, 16), lambda i: (1, i, 0)),
            pl.BlockSpec((ABN, 1), row),
            pl.BlockSpec((F_IN, H), lambda i: (0, 0)),
            pl.BlockSpec((16, H), lambda i: (0, 0)),
            pl.BlockSpec((16, H), lambda i: (0, 0)),
            pl.BlockSpec((1, H), lambda i: (0, 0)),
        ],
        out_specs=[pl.BlockSpec((ABN, 32), row)] * 8,
        out_shape=hsp + gsp,
    )(x, p1, s2p, s2p, dinv, w0, w1p, w2p, b)


def _mid_body(s0_ref, s1_ref, s2_ref, s3_ref, dinv_ref,
              p0, p1, p2, p3, g0, g1, g2, g3):
    dinv = dinv_ref[...]
    for sr, pr, gr in ((s0_ref, p0, g0), (s1_ref, p1, g1),
                       (s2_ref, p2, g2), (s3_ref, p3, g3)):
        p = sr[0] * dinv
        pr[...] = p
        gr[...] = p * dinv


def _tc_mid(sq, dinv):
    row = lambda i: (i, 0)
    qspec = [pl.BlockSpec((1, MBN, 32), (lambda q: (lambda i: (q, i, 0)))(q))
             for q in range(4)]
    osp = [jax.ShapeDtypeStruct((N, 32), jnp.float32)] * 8
    return pl.pallas_call(
        _mid_body,
        grid=(N // MBN,),
        in_specs=qspec + [pl.BlockSpec((MBN, 1), row)],
        out_specs=[pl.BlockSpec((MBN, 32), row)] * 8,
        out_shape=osp,
    )(sq, sq, sq, sq, dinv)


def _act23_body(h0r, h1r, h2r, h3r, p0r, p1r, p2r, p3r,
                s0r, s1r, s2r, s3r, dinv_ref, w0_ref, w1_ref, w2_ref, b_ref,
                h0, h1, h2, h3, g0, g1, g2, g3):
    dinv = dinv_ref[...]
    hprev = jnp.concatenate([h0r[...], h1r[...], h2r[...], h3r[...]], axis=1)
    p1 = jnp.concatenate([p0r[...], p1r[...], p2r[...], p3r[...]], axis=1)
    p2 = jnp.concatenate([s0r[0], s1r[0], s2r[0], s3r[0]], axis=1) * dinv
    acc = (jnp.dot(hprev, w0_ref[...]) + jnp.dot(p1, w1_ref[...]) +
           jnp.dot(p2, w2_ref[...]) + b_ref[...])
    h = _leaky(acc)
    g = h * dinv
    for q, (hr, gr) in enumerate(((h0, g0), (h1, g1), (h2, g2), (h3, g3))):
        hr[...] = h[:, q * 32:(q + 1) * 32]
        gr[...] = g[:, q * 32:(q + 1) * 32]


def _tc_act23(hq, p1q, s2q, dinv, w0, w1, w2, b):
    row = lambda i: (i, 0)
    qspec = [pl.BlockSpec((1, ABN, 32),
                          (lambda q: (lambda i: (q, i, 0)))(q))
             for q in range(4)]
    hsp = [jax.ShapeDtypeStruct((NROWS, 32), jnp.float32)] * 4
    gsp = [jax.ShapeDtypeStruct((N, 32), jnp.float32)] * 4
    return pl.pallas_call(
        _act23_body,
        grid=(NROWS // ABN,),
        in_specs=(
            [pl.BlockSpec((ABN, 32), row)] * 8 +
            qspec +
            [pl.BlockSpec((ABN, 1), row),
             pl.BlockSpec((H, H), lambda i: (0, 0)),
             pl.BlockSpec((H, H), lambda i: (0, 0)),
             pl.BlockSpec((H, H), lambda i: (0, 0)),
             pl.BlockSpec((1, H), lambda i: (0, 0))]),
        out_specs=[pl.BlockSpec((ABN, 32), row)] * 8,
        out_shape=hsp + gsp,
    )(*hq, *p1q, s2q, s2q, s2q, s2q, dinv, w0, w1, w2, b)


def _head_body(x1r, x2r, x3r, gref, bref, w1, b1, w2, b2, w3, b3, w4, b4,
               w5, b5, wo, bo, out_ref):
    z = jnp.concatenate([x1r[...], x2r[...], x3r[...]], axis=1)
    mu = jnp.mean(z, axis=0, keepdims=True)
    var = jnp.mean((z - mu) * (z - mu), axis=0, keepdims=True)
    z = (z - mu) / jnp.sqrt(var + EPS) * gref[...] + bref[...]
    for w, b in ((w1, b1), (w2, b2), (w3, b3), (w4, b4), (w5, b5)):
        z = _leaky(jnp.dot(z, w[...]) + b[...])
    zo = jnp.dot(z, wo[...]) + bo[...]
    out_ref[...] = jnp.concatenate([jnp.tanh(zo[:, :2]), zo[:, 2:]], axis=1)


def _tc_head(x1, x2, x3, p):
    args = [x1, x2, x3, p['bn_g'].reshape(1, H2), p['bn_b'].reshape(1, H2)]
    for i in range(1, 6):
        args += [p['l%d_W' % i], p['l%d_b' % i].reshape(1, H2)]
    args += [p['out_W'], p['out_b'].reshape(1, OUT)]
    return pl.pallas_call(
        _head_body,
        out_shape=jax.ShapeDtypeStruct((G, OUT), jnp.float32),
    )(*args)


def kernel(x, edge_index, batch, params):
    p = params
    f32 = jnp.float32

    src = edge_index[0]
    dst = edge_index[1]
    pad_e = EP - E
    src_p = jnp.concatenate([src, (jnp.arange(pad_e, dtype=jnp.int32) % N)])
    dst_p = jnp.concatenate(
        [dst, N + (jnp.arange(pad_e, dtype=jnp.int32) % (NPAD - N))])
    edges2 = jnp.stack([src_p.reshape(EPR, 128), dst_p.reshape(EPR, 128)],
                       axis=1)

    batch_p = jnp.concatenate(
        [batch, jnp.full((NB_PAD - N,), G, jnp.int32)]).reshape(NBR, 1, 128)
    batch_p = jnp.concatenate([batch_p, batch_p], axis=1)

    ones16 = jnp.ones((128, 16), f32)
    z784 = jnp.zeros((784, 32), f32)
    z16 = jnp.zeros((784, 16), f32)

    degp = _hist_deg(edges2, ones16, z16)
    bhp = _hist_batch(batch_p, ones16, z16)

    counts = (bhp[0, :G, 0] + bhp[1, :G, 0]).astype(jnp.int32)
    bounds = jnp.concatenate([
        jnp.zeros((1,), jnp.int32), jnp.cumsum(counts),
        jnp.full((272 - G - 1,), N, jnp.int32)])

    dinv, g16 = _tc_prep(x, degp)

    s1p = _hop16(g16, edges2, z16)
    p1_16, g2_16 = _tc_l1mid(s1p, dinv)
    s2p = _hop16(g2_16, edges2, z16)

    w1p = jnp.zeros((16, H), f32).at[:F_IN].set(p['c1_W'][1])
    w2p = jnp.zeros((16, H), f32).at[:F_IN].set(p['c1_W'][2])
    outs = _tc_act1(x, p1_16, s2p, dinv, p['c1_W'][0], w1p, w2p,
                    p['c1_b'].reshape(1, H))
    hq, gq = list(outs[:4]), list(outs[4:])

    xs = []
    for layer in (2, 3):
        W = p['c%d_W' % layer]
        b = p['c%d_b' % layer].reshape(1, H)
        x_pool = _pool(*hq, bounds).reshape(G, 256)
        xs.append(x_pool)
        s1q = _hop32(*gq, edges2, z784)
        outs = _tc_mid(s1q, dinv)
        p1q, g2q = list(outs[:4]), list(outs[4:])
        s2q = _hop32(*g2q, edges2, z784)
        outs = _tc_act23(hq, p1q, s2q, dinv, W[0], W[1], W[2], b)
        hq, gq = list(outs[:4]), list(outs[4:])
    xs.append(_pool(*hq, bounds).reshape(G, 256))

    return _tc_head(xs[0], xs[1], xs[2], p)


# speedup vs baseline: 9.3195x; 1.0499x over previous
import functools

import jax
import jax.numpy as jnp
from jax import lax
from jax.experimental import pallas as pl
from jax.experimental.pallas import tpu as pltpu
from jax.experimental.pallas import tpu_sc as plsc

N = 50000
E = 800000
F_IN = 7
H = 128
H2 = 6 * H
G = 256
OUT = 3
EPS = 1e-5
NEG = 0.01

NC, NS, L = 2, 16, 16
NPAD = 50176
ZR = NPAD // NS
NROWS = 50688
BN_TC = 2000
ABN = 2112
MBN = 2000

EPR = 6272
EP = EPR * 128
NB_PAD = 53248
NBR = 416
BH_ACC = 512

_mesh = plsc.VectorSubcoreMesh(core_axis_name="c", subcore_axis_name="s")
_scp = pltpu.CompilerParams(use_tc_tiling_on_sc=False,
                            needs_layout_passes=False)


def _make_hist(nrows_per_tile, acc_rows):
    zr = acc_rows // NS

    @functools.partial(
        pl.kernel, mesh=_mesh, compiler_params=_scp,
        out_type=jax.ShapeDtypeStruct((NC, acc_rows, 16), jnp.float32),
        scratch_types=[
            pltpu.VMEM((128, 16), jnp.float32),
            pltpu.VMEM((nrows_per_tile, 2, 128), jnp.int32),
            pltpu.VMEM_SHARED((acc_rows, 16), jnp.float32),
            pltpu.SemaphoreType.DMA,
        ],
    )
    def hist(ids_hbm, ones_hbm, z_hbm, out_hbm, ones_v, idx_v, acc, ssem):
        cid = lax.axis_index("c")
        sid = lax.axis_index("s")
        pltpu.sync_copy(ones_hbm, ones_v)
        nz = zr // 784 if zr % 784 == 0 else 0
        if nz:
            for k in range(nz):
                pltpu.sync_copy(
                    z_hbm, acc.at[pl.ds(sid * zr + k * 784, 784)])
        else:
            pltpu.sync_copy(z_hbm.at[pl.ds(0, zr)],
                            acc.at[pl.ds(sid * zr, zr)])
        row0 = cid * (NS * nrows_per_tile) + sid * nrows_per_tile
        pltpu.sync_copy(ids_hbm.at[pl.ds(row0, nrows_per_tile)], idx_v)
        plsc.subcore_barrier()
        if nrows_per_tile % 7 == 0:
            def body(w, _):
                d = [pltpu.async_copy(ones_v, acc.at[idx_v.at[w * 7 + j, 1]],
                                      ssem, add=True)
                     for j in range(7)]
                for x in d:
                    x.wait()
                return 0

            lax.fori_loop(0, nrows_per_tile // 7, body, 0)
        else:
            d = [pltpu.async_copy(ones_v, acc.at[idx_v.at[j, 1]], ssem,
                                  add=True)
                 for j in range(nrows_per_tile)]
            for x in d:
                x.wait()
        plsc.subcore_barrier()
        pltpu.sync_copy(acc.at[pl.ds(sid * zr, zr)],
                        out_hbm.at[cid, pl.ds(sid * zr, zr), :])

    return hist


_hist_deg = _make_hist(196, NPAD)
_hist_batch = _make_hist(13, BH_ACC)


def _edge_pass(h_hbm, e_hbm, acc, eidx, rows_v, gsem, ssem, row0, nbod):
    def body(i, _):
        pltpu.sync_copy(e_hbm.at[pl.ds(row0 + i * 7, 7)], eidx)
        g = [pltpu.async_copy(h_hbm.at[eidx.at[j, 0]],
                              rows_v.at[pl.ds(j * 128, 128)], gsem)
             for j in range(7)]
        for d in g:
            d.wait()
        s = [pltpu.async_copy(rows_v.at[pl.ds(j * 128, 128)],
                              acc.at[eidx.at[j, 1]], ssem, add=True)
             for j in range(7)]
        for d in s:
            d.wait()
        return 0

    lax.fori_loop(0, nbod, body, 0)


@functools.partial(
    pl.kernel, mesh=_mesh, compiler_params=_scp,
    out_type=jax.ShapeDtypeStruct((NC, NPAD, 16), jnp.float32),
    scratch_types=[
        pltpu.VMEM((7, 2, 128), jnp.int32),
        pltpu.VMEM((896, 16), jnp.float32),
        pltpu.VMEM_SHARED((NPAD, 16), jnp.float32),
        pltpu.SemaphoreType.DMA,
        pltpu.SemaphoreType.DMA,
    ],
)
def _hop16(h_hbm, e_hbm, z_hbm, out_hbm, eidx, rows_v, acc, gsem, ssem):
    cid = lax.axis_index("c")
    sid = lax.axis_index("s")
    for k in range(4):
        pltpu.sync_copy(z_hbm, acc.at[pl.ds(sid * ZR + k * 784, 784)])
    plsc.subcore_barrier()
    row0 = cid * (NS * 196) + sid * 196
    _edge_pass(h_hbm, e_hbm, acc, eidx, rows_v, gsem, ssem, row0, 28)
    plsc.subcore_barrier()
    pltpu.sync_copy(acc.at[pl.ds(sid * ZR, ZR)],
                    out_hbm.at[cid, pl.ds(sid * ZR, ZR), :])


@functools.partial(
    pl.kernel, mesh=_mesh, compiler_params=_scp,
    out_type=jax.ShapeDtypeStruct((4, NPAD, 32), jnp.float32),
    scratch_types=[
        pltpu.VMEM((7, 2, 128), jnp.int32),
        pltpu.VMEM((896, 32), jnp.float32),
        pltpu.VMEM_SHARED((NPAD, 32), jnp.float32),
        pltpu.SemaphoreType.DMA,
        pltpu.SemaphoreType.DMA,
    ],
)
def _hop32(h0, h1, h2, h3, e_hbm, z_hbm, out_hbm, eidx, rows_v, acc,
           gsem, ssem):
    cid = lax.axis_index("c")
    sid = lax.axis_index("s")
    hs = (h0, h1, h2, h3)
    for q in range(4):
        @pl.when(cid == q // 2)
        def _q(q=q):
            for k in range(4):
                pltpu.sync_copy(z_hbm,
                                acc.at[pl.ds(sid * ZR + k * 784, 784)])
            plsc.subcore_barrier()
            _edge_pass(hs[q], e_hbm, acc, eidx, rows_v, gsem, ssem,
                       sid * 392, 56)
            plsc.subcore_barrier()
            pltpu.sync_copy(acc.at[pl.ds(sid * ZR, ZR)],
                            out_hbm.at[q, pl.ds(sid * ZR, ZR), :])
            plsc.subcore_barrier()


RW = 256

@functools.partial(
    pl.kernel, mesh=_mesh, compiler_params=_scp,
    out_type=jax.ShapeDtypeStruct((G * 256,), jnp.float32),
    scratch_types=[
        pltpu.VMEM((RW, 32), jnp.float32),
        pltpu.VMEM((272,), jnp.int32),
        pltpu.VMEM((8 * 256,), jnp.float32),
    ],
)
def _pool(h0, h1, h2, h3, bnd_hbm, out_hbm, buf_v, bnd_v, res_v):
    cid = lax.axis_index("c")
    sid = lax.axis_index("s")
    wid = sid * NC + cid
    pltpu.sync_copy(bnd_hbm, bnd_v)
    lanes = lax.iota(jnp.int32, 16)
    g0 = wid * 8
    sv = plsc.load_gather(bnd_v, [g0 + lanes])
    ev = plsc.load_gather(bnd_v, [g0 + 1 + lanes])
    hs = (h0, h1, h2, h3)
    for gi in range(8):
        s = sv[gi]
        e = ev[gi]
        cnt = e - s
        nwin = (cnt + RW - 1) // RW
        denom = jnp.maximum(cnt.astype(jnp.float32), 1.0)
        for q in range(4):
            hq = hs[q]

            def wbody(w, carry, hq=hq, s=s, cnt=cnt):
                s0, s1, m0, m1 = carry
                ws = s + w * RW
                pltpu.sync_copy(hq.at[pl.ds(ws, RW)], buf_v)
                rem = jnp.minimum(RW, cnt - w * RW)

                def rbody(r, c):
                    a0, a1, b0, b1 = c
                    v0 = buf_v[r, pl.ds(0, 16)]
                    v1 = buf_v[r, pl.ds(16, 16)]
                    return (a0 + v0, a1 + v1,
                            jnp.maximum(b0, v0), jnp.maximum(b1, v1))

                return lax.fori_loop(0, rem, rbody, (s0, s1, m0, m1))

            init = (jnp.zeros((16,), jnp.float32),
                    jnp.zeros((16,), jnp.float32),
                    jnp.full((16,), -jnp.inf, jnp.float32),
                    jnp.full((16,), -jnp.inf, jnp.float32))
            s0, s1, m0, m1 = lax.fori_loop(0, nwin, wbody, init)
            base = gi * 256 + q * 32
            res_v[pl.ds(base, 16)] = s0 / denom
            res_v[pl.ds(base + 16, 16)] = s1 / denom
            res_v[pl.ds(base + 128, 16)] = jnp.where(cnt > 0, m0, 0.0)
            res_v[pl.ds(base + 144, 16)] = jnp.where(cnt > 0, m1, 0.0)
    pltpu.sync_copy(res_v, out_hbm.at[pl.ds(wid * 2048, 2048)])


def _leaky(x):
    return jnp.where(x > 0, x, NEG * x)


def _prep_body(x_ref, d0_ref, d1_ref, dinv_ref, g_ref):
    deg = d0_ref[0, :, 0:1] + d1_ref[0, :, 0:1]
    dinv = jnp.where(deg > 0, lax.rsqrt(jnp.maximum(deg, 1.0)), 0.0)
    dinv_ref[...] = dinv
    g_ref[...] = jnp.concatenate(
        [x_ref[...] * dinv, jnp.zeros((BN_TC, 16 - F_IN), jnp.float32)],
        axis=1)


def _tc_prep(x, degp):
    return pl.pallas_call(
        _prep_body,
        grid=(N // BN_TC,),
        in_specs=[
            pl.BlockSpec((BN_TC, F_IN), lambda i: (i, 0)),
            pl.BlockSpec((1, BN_TC, 16), lambda i: (0, i, 0)),
            pl.BlockSpec((1, BN_TC, 16), lambda i: (1, i, 0)),
        ],
        out_specs=[
            pl.BlockSpec((BN_TC, 1), lambda i: (i, 0)),
            pl.BlockSpec((BN_TC, 16), lambda i: (i, 0)),
        ],
        out_shape=[
            jax.ShapeDtypeStruct((N, 1), jnp.float32),
            jax.ShapeDtypeStruct((N, 16), jnp.float32),
        ],
    )(x, degp, degp)


def _l1mid_body(s0_ref, s1_ref, dinv_ref, p1_ref, g2_ref):
    dinv = dinv_ref[...]
    p1 = (s0_ref[0] + s1_ref[0]) * dinv
    p1_ref[...] = p1
    g2_ref[...] = p1 * dinv


def _tc_l1mid(s1p, dinv):
    return pl.pallas_call(
        _l1mid_body,
        grid=(N // BN_TC,),
        in_specs=[
            pl.BlockSpec((1, BN_TC, 16), lambda i: (0, i, 0)),
            pl.BlockSpec((1, BN_TC, 16), lambda i: (1, i, 0)),
            pl.BlockSpec((BN_TC, 1), lambda i: (i, 0)),
        ],
        out_specs=[
            pl.BlockSpec((BN_TC, 16), lambda i: (i, 0)),
            pl.BlockSpec((BN_TC, 16), lambda i: (i, 0)),
        ],
        out_shape=[
            jax.ShapeDtypeStruct((N, 16), jnp.float32),
            jax.ShapeDtypeStruct((N, 16), jnp.float32),
        ],
    )(s1p, s1p, dinv)


def _act1_body(x_ref, p1_ref, s20_ref, s21_ref, dinv_ref, w0_ref, w1_ref,
               w2_ref, b_ref, h0, h1, h2, h3, g0, g1, g2, g3):
    dinv = dinv_ref[...]
    p2 = (s20_ref[0] + s21_ref[0]) * dinv
    acc = (jnp.dot(x_ref[...], w0_ref[...]) +
           jnp.dot(p1_ref[...], w1_ref[...]) +
           jnp.dot(p2, w2_ref[...]) + b_ref[...])
    h = _leaky(acc)
    g = h * dinv
    for q, (hr, gr) in enumerate(((h0, g0), (h1, g1), (h2, g2), (h3, g3))):
        hr[...] = h[:, q * 32:(q + 1) * 32]
        gr[...] = g[:, q * 32:(q + 1) * 32]


def _tc_act1(x, p1, s2p, dinv, w0, w1p, w2p, b):
    hsp = [jax.ShapeDtypeStruct((NROWS, 32), jnp.float32)] * 4
    gsp = [jax.ShapeDtypeStruct((N, 32), jnp.float32)] * 4
    row = lambda i: (i, 0)
    return pl.pallas_call(
        _act1_body,
        grid=(NROWS // ABN,),
        in_specs=[
            pl.BlockSpec((ABN, F_IN), row),
            pl.BlockSpec((ABN, 16), row),
            pl.BlockSpec((1, ABN, 16), lambda i: (0, i, 0)),
            pl.BlockSpec((1, ABN, 16), lambda i: (1, i, 0)),
            pl.BlockSpec((ABN, 1), row),
            pl.BlockSpec((F_IN, H), lambda i: (0, 0)),
            pl.BlockSpec((16, H), lambda i: (0, 0)),
            pl.BlockSpec((16, H), lambda i: (0, 0)),
            pl.BlockSpec((1, H), lambda i: (0, 0)),
        ],
        out_specs=[pl.BlockSpec((ABN, 32), row)] * 8,
        out_shape=hsp + gsp,
    )(x, p1, s2p, s2p, dinv, w0, w1p, w2p, b)


def _mid_body(s0_ref, s1_ref, s2_ref, s3_ref, dinv_ref,
              p0, p1, p2, p3, g0, g1, g2, g3):
    dinv = dinv_ref[...]
    for sr, pr, gr in ((s0_ref, p0, g0), (s1_ref, p1, g1),
                       (s2_ref, p2, g2), (s3_ref, p3, g3)):
        p = sr[0] * dinv
        pr[...] = p
        gr[...] = p * dinv


def _tc_mid(sq, dinv):
    row = lambda i: (i, 0)
    qspec = [pl.BlockSpec((1, MBN, 32), (lambda q: (lambda i: (q, i, 0)))(q))
             for q in range(4)]
    osp = [jax.ShapeDtypeStruct((N, 32), jnp.float32)] * 8
    return pl.pallas_call(
        _mid_body,
        grid=(N // MBN,),
        in_specs=qspec + [pl.BlockSpec((MBN, 1), row)],
        out_specs=[pl.BlockSpec((MBN, 32), row)] * 8,
        out_shape=osp,
    )(sq, sq, sq, sq, dinv)


def _act23_body(h0r, h1r, h2r, h3r, p0r, p1r, p2r, p3r,
                s0r, s1r, s2r, s3r, dinv_ref, w0_ref, w1_ref, w2_ref, b_ref,
                h0, h1, h2, h3, g0, g1, g2, g3):
    dinv = dinv_ref[...]
    hprev = jnp.concatenate([h0r[...], h1r[...], h2r[...], h3r[...]], axis=1)
    p1 = jnp.concatenate([p0r[...], p1r[...], p2r[...], p3r[...]], axis=1)
    p2 = jnp.concatenate([s0r[0], s1r[0], s2r[0], s3r[0]], axis=1) * dinv
    acc = (jnp.dot(hprev, w0_ref[...]) + jnp.dot(p1, w1_ref[...]) +
           jnp.dot(p2, w2_ref[...]) + b_ref[...])
    h = _leaky(acc)
    g = h * dinv
    for q, (hr, gr) in enumerate(((h0, g0), (h1, g1), (h2, g2), (h3, g3))):
        hr[...] = h[:, q * 32:(q + 1) * 32]
        gr[...] = g[:, q * 32:(q + 1) * 32]


def _tc_act23(hq, p1q, s2q, dinv, w0, w1, w2, b):
    row = lambda i: (i, 0)
    qspec = [pl.BlockSpec((1, ABN, 32),
                          (lambda q: (lambda i: (q, i, 0)))(q))
             for q in range(4)]
    hsp = [jax.ShapeDtypeStruct((NROWS, 32), jnp.float32)] * 4
    gsp = [jax.ShapeDtypeStruct((N, 32), jnp.float32)] * 4
    return pl.pallas_call(
        _act23_body,
        grid=(NROWS // ABN,),
        in_specs=(
            [pl.BlockSpec((ABN, 32), row)] * 8 +
            qspec +
            [pl.BlockSpec((ABN, 1), row),
             pl.BlockSpec((H, H), lambda i: (0, 0)),
             pl.BlockSpec((H, H), lambda i: (0, 0)),
             pl.BlockSpec((H, H), lambda i: (0, 0)),
             pl.BlockSpec((1, H), lambda i: (0, 0))]),
        out_specs=[pl.BlockSpec((ABN, 32), row)] * 8,
        out_shape=hsp + gsp,
    )(*hq, *p1q, s2q, s2q, s2q, s2q, dinv, w0, w1, w2, b)


def _head_body(x1r, x2r, x3r, gref, bref, w1, b1, w2, b2, w3, b3, w4, b4,
               w5, b5, wo, bo, out_ref):
    z = jnp.concatenate([x1r[...], x2r[...], x3r[...]], axis=1)
    mu = jnp.mean(z, axis=0, keepdims=True)
    var = jnp.mean((z - mu) * (z - mu), axis=0, keepdims=True)
    z = (z - mu) / jnp.sqrt(var + EPS) * gref[...] + bref[...]
    for w, b in ((w1, b1), (w2, b2), (w3, b3), (w4, b4), (w5, b5)):
        z = _leaky(jnp.dot(z, w[...]) + b[...])
    zo = jnp.dot(z, wo[...]) + bo[...]
    out_ref[...] = jnp.concatenate([jnp.tanh(zo[:, :2]), zo[:, 2:]], axis=1)


def _tc_head(x1, x2, x3, p):
    args = [x1, x2, x3, p['bn_g'].reshape(1, H2), p['bn_b'].reshape(1, H2)]
    for i in range(1, 6):
        args += [p['l%d_W' % i], p['l%d_b' % i].reshape(1, H2)]
    args += [p['out_W'], p['out_b'].reshape(1, OUT)]
    return pl.pallas_call(
        _head_body,
        out_shape=jax.ShapeDtypeStruct((G, OUT), jnp.float32),
    )(*args)


def kernel(x, edge_index, batch, params):
    p = params
    f32 = jnp.float32

    src = edge_index[0]
    dst = edge_index[1]
    pad_e = EP - E
    src_p = jnp.concatenate([src, (jnp.arange(pad_e, dtype=jnp.int32) % N)])
    dst_p = jnp.concatenate(
        [dst, N + (jnp.arange(pad_e, dtype=jnp.int32) % (NPAD - N))])
    edges2 = jnp.stack([src_p.reshape(EPR, 128), dst_p.reshape(EPR, 128)],
                       axis=1)

    batch_p = jnp.concatenate(
        [batch, jnp.full((NB_PAD - N,), G, jnp.int32)]).reshape(NBR, 1, 128)
    batch_p = jnp.concatenate([batch_p, batch_p], axis=1)

    ones16 = jnp.ones((128, 16), f32)
    z784 = jnp.zeros((784, 32), f32)
    z16 = jnp.zeros((784, 16), f32)

    degp = _hist_deg(edges2, ones16, z16)
    bhp = _hist_batch(batch_p, ones16, z16)

    counts = (bhp[0, :G, 0] + bhp[1, :G, 0]).astype(jnp.int32)
    bounds = jnp.concatenate([
        jnp.zeros((1,), jnp.int32), jnp.cumsum(counts),
        jnp.full((272 - G - 1,), N, jnp.int32)])

    dinv, g16 = _tc_prep(x, degp)

    s1p = _hop16(g16, edges2, z16)
    p1_16, g2_16 = _tc_l1mid(s1p, dinv)
    s2p = _hop16(g2_16, edges2, z16)

    w1p = jnp.zeros((16, H), f32).at[:F_IN].set(p['c1_W'][1])
    w2p = jnp.zeros((16, H), f32).at[:F_IN].set(p['c1_W'][2])
    outs = _tc_act1(x, p1_16, s2p, dinv, p['c1_W'][0], w1p, w2p,
                    p['c1_b'].reshape(1, H))
    hq, gq = list(outs[:4]), list(outs[4:])

    xs = []
    for layer in (2, 3):
        W = p['c%d_W' % layer]
        b = p['c%d_b' % layer].reshape(1, H)
        x_pool = _pool(*hq, bounds).reshape(G, 256)
        xs.append(x_pool)
        s1q = _hop32(*gq, edges2, z784)
        outs = _tc_mid(s1q, dinv)
        p1q, g2q = list(outs[:4]), list(outs[4:])
        s2q = _hop32(*g2q, edges2, z784)
        outs = _tc_act23(hq, p1q, s2q, dinv, W[0], W[1], W[2], b)
        hq, gq = list(outs[:4]), list(outs[4:])
    xs.append(_pool(*hq, bounds).reshape(G, 256))

    return _tc_head(xs[0], xs[1], xs[2], p)
